# baseline (speedup 1.0000x reference)
import functools

import jax
import jax.numpy as jnp
from jax import lax
from jax.experimental import pallas as pl
from jax.experimental.pallas import tpu as pltpu
from jax.experimental.pallas import tpu_sc as plsc

EMB = 64
B = 16384
L = 50

NC = 2
NS = 16
NW = NC * NS
RPW = B // NW
CB = 16
NG = RPW // CB
IPG = CB * L
SW = 80
NSTREAM = IPG // SW
LANES = 16
CCH = EMB // LANES


def _sc_body(cpt_hbm, msk_hbm, table_hbm, out_hbm,
             cpt_v, msk_v, idx_v, rows_v, out_v, sem):
    wid = lax.axis_index("s") * NC + lax.axis_index("c")
    wbase = wid * (RPW * L)
    rbase = wid * RPW

    def group(g, carry):
        gbase = pl.multiple_of(wbase + g * IPG, 8)
        pltpu.sync_copy(cpt_hbm.at[pl.ds(gbase, IPG)], cpt_v)
        pltpu.sync_copy(msk_hbm.at[pl.ds(gbase, IPG)], msk_v)

        def prep(i, c2):
            s = pl.ds(pl.multiple_of(i * LANES, LANES), LANES)
            c = cpt_v[s]
            m = msk_v[s]
            idx_v[s] = jnp.where(m == 0, 0, c + 1)
            return c2
        lax.fori_loop(0, IPG // LANES, prep, 0)

        copies = [
            pltpu.async_copy(
                table_hbm.at[idx_v.at[pl.ds(j * SW, SW)]],
                rows_v.at[pl.ds(j * SW, SW)],
                sem,
            )
            for j in range(NSTREAM)
        ]
        for cp in copies:
            cp.wait()

        def acc_row(r, c2):
            base = r * L

            def body(l, accs):
                k = base + l
                return tuple(
                    a + rows_v[k, pl.ds(c * LANES, LANES)]
                    for c, a in enumerate(accs)
                )

            z = jnp.zeros((LANES,), jnp.float32)
            accs = lax.fori_loop(0, L, body, (z,) * CCH)
            for c in range(CCH):
                out_v[r, pl.ds(c * LANES, LANES)] = accs[c]
            return c2
        lax.fori_loop(0, CB, acc_row, 0)

        pltpu.sync_copy(out_v, out_hbm.at[pl.ds(rbase + g * CB, CB)])
        return carry

    lax.fori_loop(0, NG, group, 0)


@functools.partial(jax.jit, static_argnums=())
def _run(cpt_flat, msk_flat, table):
    mesh = plsc.VectorSubcoreMesh(core_axis_name="c", subcore_axis_name="s")
    k = pl.kernel(
        _sc_body,
        mesh=mesh,
        compiler_params=pltpu.CompilerParams(use_tc_tiling_on_sc=False),
        out_type=jax.ShapeDtypeStruct((B, EMB), jnp.float32),
        scratch_types=[
            pltpu.VMEM((IPG,), jnp.int32),
            pltpu.VMEM((IPG,), jnp.int32),
            pltpu.VMEM((IPG,), jnp.int32),
            pltpu.VMEM((IPG, EMB), jnp.float32),
            pltpu.VMEM((CB, EMB), jnp.float32),
            pltpu.SemaphoreType.DMA,
        ],
    )
    return k(cpt_flat, msk_flat, table)


def kernel(cpt_seq, cpt_seq_mask, table):
    cpt_flat = cpt_seq.reshape(-1)
    msk_flat = cpt_seq_mask.reshape(-1)
    return _run(cpt_flat, msk_flat, table)


# speedup vs baseline: 21.5270x; 21.5270x over previous
import functools

import jax
import jax.numpy as jnp
from jax import lax
from jax.experimental import pallas as pl
from jax.experimental.pallas import tpu as pltpu
from jax.experimental.pallas import tpu_sc as plsc

EMB = 64
B = 16384
L = 50

NC = 2
NS = 16
NW = NC * NS
RPW = B // NW
CB = 16
NG = RPW // CB
IPG = CB * L
SW = 80
NSTREAM = IPG // SW
LANES = 16
CCH = EMB // LANES
PAD = 8192
PAD_BASE = 100000 + 1


def _sc_body(cpt_hbm, msk_hbm, table_hbm, out_hbm,
             cpt_v, msk_v, idx_v, rows_v, out_v, sem):
    wid = lax.axis_index("s") * NC + lax.axis_index("c")
    wbase = wid * (RPW * L)
    rbase = wid * RPW

    def group(g, carry):
        gbase = pl.multiple_of(wbase + g * IPG, 8)
        pltpu.sync_copy(cpt_hbm.at[pl.ds(gbase, IPG)], cpt_v)
        pltpu.sync_copy(msk_hbm.at[pl.ds(gbase, IPG)], msk_v)

        lane = lax.iota(jnp.int32, 16)

        def prep(i, c2):
            s = pl.ds(pl.multiple_of(i * LANES, LANES), LANES)
            c = cpt_v[s]
            m = msk_v[s]
            pad_idx = PAD_BASE + ((gbase + i * LANES + lane) & (PAD - 1))
            idx_v[s] = jnp.where(m == 0, pad_idx, c + 1)
            return c2
        lax.fori_loop(0, IPG // LANES, prep, 0)

        copies = [
            pltpu.async_copy(
                table_hbm.at[idx_v.at[pl.ds(j * SW, SW)]],
                rows_v.at[pl.ds(j * SW, SW)],
                sem,
            )
            for j in range(NSTREAM)
        ]
        for cp in copies:
            cp.wait()

        def acc_row(r, c2):
            base = r * L

            def body(l, accs):
                k = base + l
                return tuple(
                    a + rows_v[k, pl.ds(c * LANES, LANES)]
                    for c, a in enumerate(accs)
                )

            z = jnp.zeros((LANES,), jnp.float32)
            accs = lax.fori_loop(0, L, body, (z,) * CCH)
            for c in range(CCH):
                out_v[r, pl.ds(c * LANES, LANES)] = accs[c]
            return c2
        lax.fori_loop(0, CB, acc_row, 0)

        pltpu.sync_copy(out_v, out_hbm.at[pl.ds(rbase + g * CB, CB)])
        return carry

    lax.fori_loop(0, NG, group, 0)


@functools.partial(jax.jit, static_argnums=())
def _run(cpt_flat, msk_flat, table):
    mesh = plsc.VectorSubcoreMesh(core_axis_name="c", subcore_axis_name="s")
    k = pl.kernel(
        _sc_body,
        mesh=mesh,
        compiler_params=pltpu.CompilerParams(use_tc_tiling_on_sc=False),
        out_type=jax.ShapeDtypeStruct((B, EMB), jnp.float32),
        scratch_types=[
            pltpu.VMEM((IPG,), jnp.int32),
            pltpu.VMEM((IPG,), jnp.int32),
            pltpu.VMEM((IPG,), jnp.int32),
            pltpu.VMEM((IPG, EMB), jnp.float32),
            pltpu.VMEM((CB, EMB), jnp.float32),
            pltpu.SemaphoreType.DMA,
        ],
    )
    return k(cpt_flat, msk_flat, table)


def kernel(cpt_seq, cpt_seq_mask, table):
    cpt_flat = cpt_seq.reshape(-1)
    msk_flat = cpt_seq_mask.reshape(-1)
    table_pad = jnp.concatenate(
        [table, jnp.zeros((PAD, EMB), table.dtype)], axis=0)
    return _run(cpt_flat, msk_flat, table_pad)


# speedup vs baseline: 23.8155x; 1.1063x over previous
import functools

import jax
import jax.numpy as jnp
from jax import lax
from jax.experimental import pallas as pl
from jax.experimental.pallas import tpu as pltpu
from jax.experimental.pallas import tpu_sc as plsc

EMB = 64
B = 16384
L = 50

NC = 2
NS = 16
NW = NC * NS
RPW = B // NW
CB = 16
NG = RPW // CB
IPG = CB * L
SW = 80
NSTREAM = IPG // SW
LANES = 16
CCH = EMB // LANES


def _sc_body(cpt_hbm, msk_hbm, table_hbm, out_hbm,
             cpt_v, msk_v, idx_v, fmsk_v, rows_v, out_v, sem):
    wid = lax.axis_index("s") * NC + lax.axis_index("c")
    wbase = wid * (RPW * L)
    rbase = wid * RPW

    def group(g, carry):
        gbase = pl.multiple_of(wbase + g * IPG, 8)
        pltpu.sync_copy(cpt_hbm.at[pl.ds(gbase, IPG)], cpt_v)
        pltpu.sync_copy(msk_hbm.at[pl.ds(gbase, IPG)], msk_v)

        def prep(i, c2):
            s = pl.ds(pl.multiple_of(i * LANES, LANES), LANES)
            c = cpt_v[s]
            m = msk_v[s]
            idx_v[s] = c + 1
            fmsk_v[s] = jnp.where(m == 0, 0.0, 1.0)
            return c2
        lax.fori_loop(0, IPG // LANES, prep, 0)

        copies = [
            pltpu.async_copy(
                table_hbm.at[idx_v.at[pl.ds(j * SW, SW)]],
                rows_v.at[pl.ds(j * SW, SW)],
                sem,
            )
            for j in range(NSTREAM)
        ]
        for cp in copies:
            cp.wait()

        def acc_row(r, c2):
            base = r * L

            def body(l, accs):
                k = base + l
                kb = pl.multiple_of((k // LANES) * LANES, LANES)
                ch = fmsk_v[pl.ds(kb, LANES)]
                mv = ch.at[lax.iota(jnp.int32, LANES) * 0 + (k - kb)].get(
                    mode="promise_in_bounds")
                return tuple(
                    a + rows_v[k, pl.ds(c * LANES, LANES)] * mv
                    for c, a in enumerate(accs)
                )

            z = jnp.zeros((LANES,), jnp.float32)
            accs = lax.fori_loop(0, L, body, (z,) * CCH)
            for c in range(CCH):
                out_v[r, pl.ds(c * LANES, LANES)] = accs[c]
            return c2
        lax.fori_loop(0, CB, acc_row, 0)

        pltpu.sync_copy(out_v, out_hbm.at[pl.ds(rbase + g * CB, CB)])
        return carry

    lax.fori_loop(0, NG, group, 0)


@functools.partial(jax.jit, static_argnums=())
def _run(cpt_flat, msk_flat, table):
    mesh = plsc.VectorSubcoreMesh(core_axis_name="c", subcore_axis_name="s")
    k = pl.kernel(
        _sc_body,
        mesh=mesh,
        compiler_params=pltpu.CompilerParams(use_tc_tiling_on_sc=False),
        out_type=jax.ShapeDtypeStruct((B, EMB), jnp.float32),
        scratch_types=[
            pltpu.VMEM((IPG,), jnp.int32),
            pltpu.VMEM((IPG,), jnp.int32),
            pltpu.VMEM((IPG,), jnp.int32),
            pltpu.VMEM((IPG,), jnp.float32),
            pltpu.VMEM((IPG, EMB), jnp.float32),
            pltpu.VMEM((CB, EMB), jnp.float32),
            pltpu.SemaphoreType.DMA,
        ],
    )
    return k(cpt_flat, msk_flat, table)


def kernel(cpt_seq, cpt_seq_mask, table):
    cpt_flat = cpt_seq.reshape(-1)
    msk_flat = cpt_seq_mask.reshape(-1)
    return _run(cpt_flat, msk_flat, table)


# speedup vs baseline: 39.4370x; 1.6559x over previous
import functools

import jax
import jax.numpy as jnp
from jax import lax
from jax.experimental import pallas as pl
from jax.experimental.pallas import tpu as pltpu
from jax.experimental.pallas import tpu_sc as plsc

EMB = 64
B = 16384
L = 50

NC = 2
NS = 16
NW = NC * NS
RPW = B // NW
CB = 16
NG = RPW // CB
IPG = CB * L
SW = 80
NSTREAM = IPG // SW
LANES = 16
CCH = EMB // LANES


def _sc_body(cpt_hbm, msk_hbm, table_hbm, out_hbm,
             cpt_v0, cpt_v1, msk_v0, msk_v1, idx_v0, idx_v1,
             fmsk_v0, fmsk_v1, rows_v0, rows_v1, out_v0, out_v1,
             sem_ld0, sem_ld1, sem_g0, sem_g1, sem_o0, sem_o1):
    cpt_v = (cpt_v0, cpt_v1)
    msk_v = (msk_v0, msk_v1)
    idx_v = (idx_v0, idx_v1)
    fmsk_v = (fmsk_v0, fmsk_v1)
    rows_v = (rows_v0, rows_v1)
    out_v = (out_v0, out_v1)
    sem_ld = (sem_ld0, sem_ld1)
    sem_g = (sem_g0, sem_g1)
    sem_o = (sem_o0, sem_o1)

    wid = lax.axis_index("s") * NC + lax.axis_index("c")
    wbase = wid * (RPW * L)
    rbase = wid * RPW

    def gb(g):
        return pl.multiple_of(wbase + g * IPG, 8)

    def prep(g, b):
        def step(i, c2):
            s = pl.ds(pl.multiple_of(i * LANES, LANES), LANES)
            c = cpt_v[b][s]
            m = msk_v[b][s]
            idx_v[b][s] = c + 1
            fmsk_v[b][s] = jnp.where(m == 0, 0.0, 1.0)
            return c2
        lax.fori_loop(0, IPG // LANES, step, 0)

    def fire_gathers(b):
        for j in range(NSTREAM):
            pltpu.make_async_copy(
                table_hbm.at[idx_v[b].at[pl.ds(j * SW, SW)]],
                rows_v[b].at[pl.ds(j * SW, SW)],
                sem_g[b],
            ).start()

    def wait_gathers(b):
        pltpu.make_async_copy(
            table_hbm.at[idx_v[b]], rows_v[b], sem_g[b],
        ).wait()

    def fire_loads(g, b):
        pltpu.make_async_copy(
            cpt_hbm.at[pl.ds(gb(g), IPG)], cpt_v[b], sem_ld[b]).start()
        pltpu.make_async_copy(
            msk_hbm.at[pl.ds(gb(g), IPG)], msk_v[b], sem_ld[b]).start()

    def wait_loads(g, b):
        pltpu.make_async_copy(
            cpt_hbm.at[pl.ds(gb(g), IPG)], cpt_v[b], sem_ld[b]).wait()
        pltpu.make_async_copy(
            msk_hbm.at[pl.ds(gb(g), IPG)], msk_v[b], sem_ld[b]).wait()

    def accumulate(b):
        def acc_row(r, c2):
            base = r * L

            def chunk(k, accs):
                kb = pl.multiple_of((k // LANES) * LANES, LANES)
                ch = fmsk_v[b][pl.ds(kb, LANES)]
                mv = ch.at[lax.iota(jnp.int32, LANES) * 0 + (k - kb)].get(
                    mode="promise_in_bounds")
                return tuple(
                    a + rows_v[b][k, pl.ds(c * LANES, LANES)] * mv
                    for c, a in enumerate(accs)
                )

            def body(l2, accs):
                return chunk(base + l2 * 2 + 1, chunk(base + l2 * 2, accs))

            z = jnp.zeros((LANES,), jnp.float32)
            accs = lax.fori_loop(0, L // 2, body, (z,) * CCH)
            for c in range(CCH):
                out_v[b][r, pl.ds(c * LANES, LANES)] = accs[c]
            return c2
        lax.fori_loop(0, CB, acc_row, 0)

    def fire_out(g, b):
        pltpu.make_async_copy(
            out_v[b], out_hbm.at[pl.ds(rbase + g * CB, CB)], sem_o[b]).start()

    def wait_out(g, b):
        pltpu.make_async_copy(
            out_v[b], out_hbm.at[pl.ds(rbase + g * CB, CB)], sem_o[b]).wait()

    fire_loads(0, 0)
    wait_loads(0, 0)
    prep(0, 0)
    fire_gathers(0)
    fire_loads(1, 1)

    def pair(g2, carry):
        for bpar in (0, 1):
            g = g2 * 2 + bpar

            @pl.when(g + 1 < NG)
            def _():
                wait_loads(g + 1, 1 - bpar)
                prep(g + 1, 1 - bpar)
                fire_gathers(1 - bpar)

            @pl.when(g + 2 < NG)
            def _():
                fire_loads(g + 2, bpar)

            wait_gathers(bpar)

            @pl.when(g >= 2)
            def _():
                wait_out(g - 2, bpar)

            accumulate(bpar)
            fire_out(g, bpar)
        return carry

    lax.fori_loop(0, NG // 2, pair, 0)
    wait_out(NG - 2, 0)
    wait_out(NG - 1, 1)


@functools.partial(jax.jit, static_argnums=())
def _run(cpt_flat, msk_flat, table):
    mesh = plsc.VectorSubcoreMesh(core_axis_name="c", subcore_axis_name="s")
    k = pl.kernel(
        _sc_body,
        mesh=mesh,
        compiler_params=pltpu.CompilerParams(use_tc_tiling_on_sc=False),
        out_type=jax.ShapeDtypeStruct((B, EMB), jnp.float32),
        scratch_types=[
            pltpu.VMEM((IPG,), jnp.int32),
            pltpu.VMEM((IPG,), jnp.int32),
            pltpu.VMEM((IPG,), jnp.int32),
            pltpu.VMEM((IPG,), jnp.int32),
            pltpu.VMEM((IPG,), jnp.int32),
            pltpu.VMEM((IPG,), jnp.int32),
            pltpu.VMEM((IPG,), jnp.float32),
            pltpu.VMEM((IPG,), jnp.float32),
            pltpu.VMEM((IPG, EMB), jnp.float32),
            pltpu.VMEM((IPG, EMB), jnp.float32),
            pltpu.VMEM((CB, EMB), jnp.float32),
            pltpu.VMEM((CB, EMB), jnp.float32),
            pltpu.SemaphoreType.DMA,
            pltpu.SemaphoreType.DMA,
            pltpu.SemaphoreType.DMA,
            pltpu.SemaphoreType.DMA,
            pltpu.SemaphoreType.DMA,
            pltpu.SemaphoreType.DMA,
        ],
    )
    return k(cpt_flat, msk_flat, table)


def kernel(cpt_seq, cpt_seq_mask, table):
    cpt_flat = cpt_seq.reshape(-1)
    msk_flat = cpt_seq_mask.reshape(-1)
    return _run(cpt_flat, msk_flat, table)
